# Initial kernel scaffold; baseline (speedup 1.0000x reference)
import jax
import jax.numpy as jnp
from jax.experimental import pallas as pl


def kernel(x, edge_index, W1, b1, W2, b2, W3, b3):
    raise NotImplementedError("write your pallas kernel here")


# speedup vs baseline: 1.7862x; 1.7862x over previous
import functools

import jax
import jax.numpy as jnp
from jax import lax
from jax.experimental import pallas as pl
from jax.experimental.pallas import tpu as pltpu
from jax.experimental.pallas import tpu_sc as plsc

N = 10000
NP = 10240
E = 320000
EP = 327680
D = 128
H = 64

NW = 32
CHUNK = 128
NCH = EP // NW // CHUNK


@functools.cache
def _mesh():
    return plsc.VectorSubcoreMesh(core_axis_name="c", subcore_axis_name="s")


@functools.cache
def _make_agg(C: int):
    @functools.partial(
        pl.kernel,
        mesh=_mesh(),
        out_type=jax.ShapeDtypeStruct((2, NP, C), jnp.float32),
        scratch_types=[
            pltpu.VMEM((NCH, CHUNK), jnp.int32),
            pltpu.VMEM((NCH, CHUNK), jnp.int32),
            pltpu.VMEM((CHUNK, C), jnp.float32),
            pltpu.VMEM((CHUNK,), jnp.int32),
            pltpu.VMEM_SHARED((NP, C), jnp.float32),
        ],
    )
    def agg(x_hbm, src_hbm, dst_hbm, zeros_hbm, out_hbm,
            sidx, didx, rows, dcur, acc):
        c = lax.axis_index("c")
        s = lax.axis_index("s")
        w = c * 16 + s
        pltpu.sync_copy(src_hbm.at[w], sidx)
        pltpu.sync_copy(dst_hbm.at[w], didx)

        @pl.when(s == 0)
        def _():
            pltpu.sync_copy(zeros_hbm, acc)

        plsc.subcore_barrier()

        def body(i, carry):
            pltpu.sync_copy(x_hbm.at[sidx.at[i]], rows)
            for j in range(CHUNK // 16):
                dcur[pl.ds(j * 16, 16)] = didx[i, pl.ds(j * 16, 16)]
            pltpu.sync_copy(rows, acc.at[dcur], add=True)
            return carry

        lax.fori_loop(0, NCH, body, 0)
        plsc.subcore_barrier()

        @pl.when(s == 0)
        def _():
            pltpu.sync_copy(acc, out_hbm.at[c])

    return agg


@functools.cache
def _make_deg():
    @functools.partial(
        pl.kernel,
        mesh=_mesh(),
        out_type=(jax.ShapeDtypeStruct((2, NP, 16), jnp.float32),
                  jax.ShapeDtypeStruct((2, NP, 16), jnp.float32)),
        scratch_types=[
            pltpu.VMEM((NCH, CHUNK), jnp.int32),
            pltpu.VMEM((NCH, CHUNK), jnp.int32),
            pltpu.VMEM((CHUNK, 16), jnp.float32),
            pltpu.VMEM((CHUNK,), jnp.int32),
            pltpu.VMEM_SHARED((NP, 16), jnp.float32),
            pltpu.VMEM_SHARED((NP, 16), jnp.float32),
        ],
    )
    def deg(src_hbm, dst_hbm, zeros_hbm, ones_hbm, outdeg_hbm, indeg_hbm,
            sidx, didx, ones, cur, acc_o, acc_i):
        c = lax.axis_index("c")
        s = lax.axis_index("s")
        w = c * 16 + s
        pltpu.sync_copy(src_hbm.at[w], sidx)
        pltpu.sync_copy(dst_hbm.at[w], didx)
        pltpu.sync_copy(ones_hbm, ones)

        @pl.when(s == 0)
        def _():
            pltpu.sync_copy(zeros_hbm, acc_o)
            pltpu.sync_copy(zeros_hbm, acc_i)

        plsc.subcore_barrier()

        def body(i, carry):
            for j in range(CHUNK // 16):
                cur[pl.ds(j * 16, 16)] = sidx[i, pl.ds(j * 16, 16)]
            pltpu.sync_copy(ones, acc_o.at[cur], add=True)
            for j in range(CHUNK // 16):
                cur[pl.ds(j * 16, 16)] = didx[i, pl.ds(j * 16, 16)]
            pltpu.sync_copy(ones, acc_i.at[cur], add=True)
            return carry

        lax.fori_loop(0, NCH, body, 0)
        plsc.subcore_barrier()

        @pl.when(s == 0)
        def _():
            pltpu.sync_copy(acc_o, outdeg_hbm.at[c])
            pltpu.sync_copy(acc_i, indeg_hbm.at[c])

    return deg


_RB = 512


def _nrm(pa, pb):
    deg = pa[:, 0:1] + pb[:, 0:1]
    return lax.rsqrt(jnp.maximum(deg, 1.0))


def _tc0_body(x, doa, dob, o):
    o[...] = x[...] * _nrm(doa, dob)


def _tc1_body(aa, ab, dia, dib, doa, dob, w, b, o):
    agg = aa[...] + ab[...]
    z = jnp.dot(agg, w[...], preferred_element_type=jnp.float32)
    z = z * _nrm(dia, dib) + b[...]
    z = jnp.maximum(z, 0.0)
    o[...] = z * _nrm(doa, dob)


def _tc3_body(aa, ab, dia, dib, w3, b, o):
    agg = aa[...] + ab[...]
    z = jnp.dot(agg, w3[...], preferred_element_type=jnp.float32)
    o[...] = z * _nrm(dia, dib) + b[...]


def _row_spec(cols):
    return pl.BlockSpec((_RB, cols), lambda i: (i, 0))


def _full_spec(shape):
    return pl.BlockSpec(shape, lambda i: tuple(0 for _ in shape))


def _tc_call(body, in_specs, out_cols, args):
    return pl.pallas_call(
        body,
        grid=(NP // _RB,),
        in_specs=in_specs,
        out_specs=_row_spec(out_cols),
        out_shape=jax.ShapeDtypeStruct((NP, out_cols), jnp.float32),
    )(*args)


def kernel(x, edge_index, W1, b1, W2, b2, W3, b3):
    pad = jnp.full((EP - E,), N, jnp.int32)
    src = jnp.concatenate([edge_index[0], pad]).reshape(NW, NCH, CHUNK)
    dst = jnp.concatenate([edge_index[1], pad]).reshape(NW, NCH, CHUNK)
    xp = jnp.pad(x, ((0, NP - N), (0, 0)))
    zeros128 = jnp.zeros((NP, 128), jnp.float32)
    zeros16 = jnp.zeros((NP, 16), jnp.float32)
    ones16 = jnp.ones((CHUNK, 16), jnp.float32)
    b1r = b1.reshape(1, 2 * H)
    b2r = b2.reshape(1, 2 * H)
    b3r = b3.reshape(1, H)

    ones_np = jnp.ones((NP, 128), jnp.float32)
    di_parts = _make_agg(128)(ones_np, src, dst, zeros128)
    do_parts = _make_agg(128)(ones_np, dst, src, zeros128)
    dia, dib = di_parts[0, :, :16], di_parts[1, :, :16]
    doa, dob = do_parts[0, :, :16], do_parts[1, :, :16]

    d16 = _row_spec(16)
    t0 = _tc_call(_tc0_body, [_row_spec(128), d16, d16], 128,
                  (xp, doa, dob))
    a1 = _make_agg(128)(t0, src, dst, zeros128)
    t1 = _tc_call(
        _tc1_body,
        [_row_spec(128), _row_spec(128), d16, d16, d16, d16,
         _full_spec((128, 128)), _full_spec((1, 128))],
        128, (a1[0], a1[1], dia, dib, doa, dob, W1, b1r))
    a2 = _make_agg(128)(t1, src, dst, zeros128)
    t2 = _tc_call(
        _tc1_body,
        [_row_spec(128), _row_spec(128), d16, d16, d16, d16,
         _full_spec((128, 128)), _full_spec((1, 128))],
        128, (a2[0], a2[1], dia, dib, doa, dob, W2, b2r))
    a3 = _make_agg(128)(t2, src, dst, zeros128)
    out = _tc_call(
        _tc3_body,
        [_row_spec(128), _row_spec(128), d16, d16,
         _full_spec((128, 64)), _full_spec((1, 64))],
        64, (a3[0], a3[1], dia, dib, W3, b3r))
    return out[:N]


# speedup vs baseline: 2.4786x; 1.3876x over previous
import functools

import jax
import jax.numpy as jnp
from jax import lax
from jax.experimental import pallas as pl
from jax.experimental.pallas import tpu as pltpu
from jax.experimental.pallas import tpu_sc as plsc

N = 10000
NP = 10240
E = 320000
EP = 327680
D = 128
H = 64

NW = 32
CHUNK = 128
NCH = EP // NW // CHUNK


@functools.cache
def _mesh():
    return plsc.VectorSubcoreMesh(core_axis_name="c", subcore_axis_name="s")


@functools.cache
def _make_agg(C: int):
    @functools.partial(
        pl.kernel,
        mesh=_mesh(),
        out_type=jax.ShapeDtypeStruct((2, NP, C), jnp.float32),
        scratch_types=[
            pltpu.VMEM((NCH, CHUNK), jnp.int32),
            pltpu.VMEM((NCH, CHUNK), jnp.int32),
            pltpu.VMEM((CHUNK, C), jnp.float32),
            pltpu.VMEM((CHUNK,), jnp.int32),
            pltpu.VMEM_SHARED((NP, C), jnp.float32),
        ],
    )
    def agg(x_hbm, src_hbm, dst_hbm, zeros_hbm, out_hbm,
            sidx, didx, rows, dcur, acc):
        c = lax.axis_index("c")
        s = lax.axis_index("s")
        w = c * 16 + s
        pltpu.sync_copy(src_hbm.at[w], sidx)
        pltpu.sync_copy(dst_hbm.at[w], didx)

        @pl.when(s == 0)
        def _():
            pltpu.sync_copy(zeros_hbm, acc)

        plsc.subcore_barrier()

        def body(i, carry):
            pltpu.sync_copy(x_hbm.at[sidx.at[i]], rows)
            for j in range(CHUNK // 16):
                dcur[pl.ds(j * 16, 16)] = didx[i, pl.ds(j * 16, 16)]
            pltpu.sync_copy(rows, acc.at[dcur], add=True)
            return carry

        lax.fori_loop(0, NCH, body, 0)
        plsc.subcore_barrier()

        @pl.when(s == 0)
        def _():
            pltpu.sync_copy(acc, out_hbm.at[c])

    return agg


@functools.cache
def _make_cnt():
    @functools.partial(
        pl.kernel,
        mesh=_mesh(),
        out_type=jax.ShapeDtypeStruct((2, NP, 128), jnp.float32),
        scratch_types=[
            pltpu.VMEM((NCH, CHUNK), jnp.int32),
            pltpu.VMEM((CHUNK, 128), jnp.float32),
            pltpu.VMEM((CHUNK,), jnp.int32),
            pltpu.VMEM_SHARED((NP, 128), jnp.float32),
        ],
    )
    def cnt(ones_hbm, idx_hbm, zeros_hbm, out_hbm, idx, rows, dcur, acc):
        c = lax.axis_index("c")
        s = lax.axis_index("s")
        w = c * 16 + s
        pltpu.sync_copy(idx_hbm.at[w], idx)
        pltpu.sync_copy(ones_hbm, rows)

        @pl.when(s == 0)
        def _():
            pltpu.sync_copy(zeros_hbm, acc)

        plsc.subcore_barrier()

        def body(i, carry):
            for j in range(CHUNK // 16):
                dcur[pl.ds(j * 16, 16)] = idx[i, pl.ds(j * 16, 16)]
            pltpu.sync_copy(rows, acc.at[dcur], add=True)
            return carry

        lax.fori_loop(0, NCH, body, 0)
        plsc.subcore_barrier()

        @pl.when(s == 0)
        def _():
            pltpu.sync_copy(acc, out_hbm.at[c])

    return cnt


@functools.cache
def _make_deg():
    @functools.partial(
        pl.kernel,
        mesh=_mesh(),
        out_type=(jax.ShapeDtypeStruct((2, NP, 16), jnp.float32),
                  jax.ShapeDtypeStruct((2, NP, 16), jnp.float32)),
        scratch_types=[
            pltpu.VMEM((NCH, CHUNK), jnp.int32),
            pltpu.VMEM((NCH, CHUNK), jnp.int32),
            pltpu.VMEM((CHUNK, 16), jnp.float32),
            pltpu.VMEM((CHUNK,), jnp.int32),
            pltpu.VMEM_SHARED((NP, 16), jnp.float32),
            pltpu.VMEM_SHARED((NP, 16), jnp.float32),
        ],
    )
    def deg(src_hbm, dst_hbm, zeros_hbm, ones_hbm, outdeg_hbm, indeg_hbm,
            sidx, didx, ones, cur, acc_o, acc_i):
        c = lax.axis_index("c")
        s = lax.axis_index("s")
        w = c * 16 + s
        pltpu.sync_copy(src_hbm.at[w], sidx)
        pltpu.sync_copy(dst_hbm.at[w], didx)
        pltpu.sync_copy(ones_hbm, ones)

        @pl.when(s == 0)
        def _():
            pltpu.sync_copy(zeros_hbm, acc_o)
            pltpu.sync_copy(zeros_hbm, acc_i)

        plsc.subcore_barrier()

        def body(i, carry):
            for j in range(CHUNK // 16):
                cur[pl.ds(j * 16, 16)] = sidx[i, pl.ds(j * 16, 16)]
            pltpu.sync_copy(ones, acc_o.at[cur], add=True)
            for j in range(CHUNK // 16):
                cur[pl.ds(j * 16, 16)] = didx[i, pl.ds(j * 16, 16)]
            pltpu.sync_copy(ones, acc_i.at[cur], add=True)
            return carry

        lax.fori_loop(0, NCH, body, 0)
        plsc.subcore_barrier()

        @pl.when(s == 0)
        def _():
            pltpu.sync_copy(acc_o, outdeg_hbm.at[c])
            pltpu.sync_copy(acc_i, indeg_hbm.at[c])

    return deg


_RB = 512


def _nrm(pa, pb):
    deg = pa[:, 0:1] + pb[:, 0:1]
    return lax.rsqrt(jnp.maximum(deg, 1.0))


def _tc0_body(x, doa, dob, o):
    o[...] = x[...] * _nrm(doa, dob)


def _tc1_body(aa, ab, dia, dib, doa, dob, w, b, o):
    agg = aa[...] + ab[...]
    z = jnp.dot(agg, w[...], preferred_element_type=jnp.float32)
    z = z * _nrm(dia, dib) + b[...]
    z = jnp.maximum(z, 0.0)
    o[...] = z * _nrm(doa, dob)


def _tc3_body(aa, ab, dia, dib, w3, b, o):
    agg = aa[...] + ab[...]
    z = jnp.dot(agg, w3[...], preferred_element_type=jnp.float32)
    o[...] = z * _nrm(dia, dib) + b[...]


def _row_spec(cols):
    return pl.BlockSpec((_RB, cols), lambda i: (i, 0))


def _full_spec(shape):
    return pl.BlockSpec(shape, lambda i: tuple(0 for _ in shape))


def _tc_call(body, in_specs, out_cols, args):
    return pl.pallas_call(
        body,
        grid=(NP // _RB,),
        in_specs=in_specs,
        out_specs=_row_spec(out_cols),
        out_shape=jax.ShapeDtypeStruct((NP, out_cols), jnp.float32),
    )(*args)


def kernel(x, edge_index, W1, b1, W2, b2, W3, b3):
    pad = jnp.full((EP - E,), N, jnp.int32)
    src = jnp.concatenate([edge_index[0], pad]).reshape(NW, NCH, CHUNK)
    dst = jnp.concatenate([edge_index[1], pad]).reshape(NW, NCH, CHUNK)
    xp = jnp.pad(x, ((0, NP - N), (0, 0)))
    zeros128 = jnp.zeros((NP, 128), jnp.float32)
    zeros16 = jnp.zeros((NP, 16), jnp.float32)
    ones16 = jnp.ones((CHUNK, 16), jnp.float32)
    b1r = b1.reshape(1, 2 * H)
    b2r = b2.reshape(1, 2 * H)
    b3r = b3.reshape(1, H)

    ones_chunk = jnp.ones((CHUNK, 128), jnp.float32)
    di_parts = _make_cnt()(ones_chunk, dst, zeros128)
    do_parts = _make_cnt()(ones_chunk, src, zeros128)
    dia, dib = di_parts[0, :, :16], di_parts[1, :, :16]
    doa, dob = do_parts[0, :, :16], do_parts[1, :, :16]

    d16 = _row_spec(16)
    t0 = _tc_call(_tc0_body, [_row_spec(128), d16, d16], 128,
                  (xp, doa, dob))
    a1 = _make_agg(128)(t0, src, dst, zeros128)
    t1 = _tc_call(
        _tc1_body,
        [_row_spec(128), _row_spec(128), d16, d16, d16, d16,
         _full_spec((128, 128)), _full_spec((1, 128))],
        128, (a1[0], a1[1], dia, dib, doa, dob, W1, b1r))
    a2 = _make_agg(128)(t1, src, dst, zeros128)
    t2 = _tc_call(
        _tc1_body,
        [_row_spec(128), _row_spec(128), d16, d16, d16, d16,
         _full_spec((128, 128)), _full_spec((1, 128))],
        128, (a2[0], a2[1], dia, dib, doa, dob, W2, b2r))
    a3 = _make_agg(128)(t2, src, dst, zeros128)
    out = _tc_call(
        _tc3_body,
        [_row_spec(128), _row_spec(128), d16, d16,
         _full_spec((128, 64)), _full_spec((1, 64))],
        64, (a3[0], a3[1], dia, dib, W3, b3r))
    return out[:N]


# speedup vs baseline: 2.4797x; 1.0004x over previous
import functools

import jax
import jax.numpy as jnp
from jax import lax
from jax.experimental import pallas as pl
from jax.experimental.pallas import tpu as pltpu
from jax.experimental.pallas import tpu_sc as plsc

N = 10000
NP = 10240
E = 320000
EP = 327680
D = 128
H = 64

NW = 32
CHUNK = 128
NCH = EP // NW // CHUNK


@functools.cache
def _mesh():
    return plsc.VectorSubcoreMesh(core_axis_name="c", subcore_axis_name="s")


@functools.cache
def _make_agg(C: int):
    @functools.partial(
        pl.kernel,
        mesh=_mesh(),
        out_type=jax.ShapeDtypeStruct((2, NP, C), jnp.float32),
        scratch_types=[
            pltpu.VMEM((NCH, CHUNK), jnp.int32),
            pltpu.VMEM((NCH, CHUNK), jnp.int32),
            pltpu.VMEM((CHUNK, C), jnp.float32),
            pltpu.VMEM((CHUNK,), jnp.int32),
            pltpu.VMEM_SHARED((NP, C), jnp.float32),
        ],
    )
    def agg(x_hbm, src_hbm, dst_hbm, zeros_hbm, out_hbm,
            sidx, didx, rows, dcur, acc):
        c = lax.axis_index("c")
        s = lax.axis_index("s")
        w = c * 16 + s
        pltpu.sync_copy(src_hbm.at[w], sidx)
        pltpu.sync_copy(dst_hbm.at[w], didx)

        @pl.when(s == 0)
        def _():
            pltpu.sync_copy(zeros_hbm, acc)

        plsc.subcore_barrier()

        def body(i, carry):
            pltpu.sync_copy(x_hbm.at[sidx.at[i]], rows)
            for j in range(CHUNK // 16):
                dcur[pl.ds(j * 16, 16)] = didx[i, pl.ds(j * 16, 16)]
            pltpu.sync_copy(rows, acc.at[dcur], add=True)
            return carry

        lax.fori_loop(0, NCH, body, 0)
        plsc.subcore_barrier()

        @pl.when(s == 0)
        def _():
            pltpu.sync_copy(acc, out_hbm.at[c])

    return agg


@functools.cache
def _make_cnt():
    @functools.partial(
        pl.kernel,
        mesh=_mesh(),
        out_type=jax.ShapeDtypeStruct((2, NP, 128), jnp.float32),
        scratch_types=[
            pltpu.VMEM((NCH, CHUNK), jnp.int32),
            pltpu.VMEM((CHUNK, 128), jnp.float32),
            pltpu.VMEM((CHUNK,), jnp.int32),
            pltpu.VMEM_SHARED((NP, 128), jnp.float32),
        ],
    )
    def cnt(ones_hbm, idx_hbm, zeros_hbm, out_hbm, idx, rows, dcur, acc):
        c = lax.axis_index("c")
        s = lax.axis_index("s")
        w = c * 16 + s
        pltpu.sync_copy(idx_hbm.at[w], idx)
        pltpu.sync_copy(ones_hbm, rows)

        @pl.when(s == 0)
        def _():
            pltpu.sync_copy(zeros_hbm, acc)

        plsc.subcore_barrier()

        def body(i, carry):
            for j in range(CHUNK // 16):
                dcur[pl.ds(j * 16, 16)] = idx[i, pl.ds(j * 16, 16)]
            pltpu.sync_copy(rows, acc.at[dcur], add=True)
            return carry

        lax.fori_loop(0, NCH, body, 0)
        plsc.subcore_barrier()

        @pl.when(s == 0)
        def _():
            pltpu.sync_copy(acc, out_hbm.at[c])

    return cnt


_RB = 512


def _nrm(pa, pb):
    deg = pa[:, 0:1] + pb[:, 0:1]
    return lax.rsqrt(jnp.maximum(deg, 1.0))


def _tc0_body(x, doa, dob, o):
    o[...] = x[...] * _nrm(doa, dob)


def _tc1_body(aa, ab, dia, dib, doa, dob, w, b, o):
    agg = aa[...] + ab[...]
    z = jnp.dot(agg, w[...], preferred_element_type=jnp.float32)
    z = z * _nrm(dia, dib) + b[...]
    z = jnp.maximum(z, 0.0)
    o[...] = z * _nrm(doa, dob)


def _tc3_body(aa, ab, dia, dib, w3, b, o):
    agg = aa[...] + ab[...]
    z = jnp.dot(agg, w3[...], preferred_element_type=jnp.float32)
    o[...] = z * _nrm(dia, dib) + b[...]


def _row_spec(cols):
    return pl.BlockSpec((_RB, cols), lambda i: (i, 0))


def _full_spec(shape):
    return pl.BlockSpec(shape, lambda i: tuple(0 for _ in shape))


def _tc_call(body, in_specs, out_cols, args):
    return pl.pallas_call(
        body,
        grid=(NP // _RB,),
        in_specs=in_specs,
        out_specs=_row_spec(out_cols),
        out_shape=jax.ShapeDtypeStruct((NP, out_cols), jnp.float32),
    )(*args)


def kernel(x, edge_index, W1, b1, W2, b2, W3, b3):
    pad = jnp.full((EP - E,), N, jnp.int32)
    src = jnp.concatenate([edge_index[0], pad]).reshape(NW, NCH, CHUNK)
    dst = jnp.concatenate([edge_index[1], pad]).reshape(NW, NCH, CHUNK)
    xp = jnp.pad(x, ((0, NP - N), (0, 0)))
    zeros128 = jnp.zeros((NP, 128), jnp.float32)
    b1r = b1.reshape(1, 2 * H)
    b2r = b2.reshape(1, 2 * H)
    b3r = b3.reshape(1, H)

    ones_chunk = jnp.ones((CHUNK, 128), jnp.float32)
    di_parts = _make_cnt()(ones_chunk, dst, zeros128)
    do_parts = _make_cnt()(ones_chunk, src, zeros128)
    dia, dib = di_parts[0, :, :16], di_parts[1, :, :16]
    doa, dob = do_parts[0, :, :16], do_parts[1, :, :16]

    d16 = _row_spec(16)
    t0 = _tc_call(_tc0_body, [_row_spec(128), d16, d16], 128,
                  (xp, doa, dob))
    a1 = _make_agg(128)(t0, src, dst, zeros128)
    t1 = _tc_call(
        _tc1_body,
        [_row_spec(128), _row_spec(128), d16, d16, d16, d16,
         _full_spec((128, 128)), _full_spec((1, 128))],
        128, (a1[0], a1[1], dia, dib, doa, dob, W1, b1r))
    a2 = _make_agg(128)(t1, src, dst, zeros128)
    t2 = _tc_call(
        _tc1_body,
        [_row_spec(128), _row_spec(128), d16, d16, d16, d16,
         _full_spec((128, 128)), _full_spec((1, 128))],
        128, (a2[0], a2[1], dia, dib, doa, dob, W2, b2r))
    a3 = _make_agg(128)(t2, src, dst, zeros128)
    out = _tc_call(
        _tc3_body,
        [_row_spec(128), _row_spec(128), d16, d16,
         _full_spec((128, 64)), _full_spec((1, 64))],
        64, (a3[0], a3[1], dia, dib, W3, b3r))
    return out[:N]
